# Initial kernel scaffold; baseline (speedup 1.0000x reference)
import jax
import jax.numpy as jnp
from jax.experimental import pallas as pl


def kernel(x, Wg, bg, W1, b1, W2, b2):
    raise NotImplementedError("write your pallas kernel here")


# speedup vs baseline: 1.5472x; 1.5472x over previous
import functools

import jax
import jax.numpy as jnp
from jax.experimental import pallas as pl


def _gating_body(x_ref, wg_ref, bg_ref, probs_ref, idx_ref, comb_ref):
    x = x_ref[...]
    logits = jnp.dot(x, wg_ref[...], preferred_element_type=jnp.float32)
    logits = logits + bg_ref[...]
    m1 = jnp.max(logits, axis=1, keepdims=True)
    i1 = jnp.argmax(logits, axis=1)
    col = jax.lax.broadcasted_iota(jnp.int32, logits.shape, 1)
    neg_inf = jnp.float32(-jnp.inf)
    masked = jnp.where(col == i1[:, None], neg_inf, logits)
    m2 = jnp.max(masked, axis=1, keepdims=True)
    i2 = jnp.argmax(masked, axis=1)

    ex = jnp.exp(logits - m1)
    probs_ref[...] = ex / jnp.sum(ex, axis=1, keepdims=True)
    idx_ref[...] = jnp.stack([i1, i2], axis=1)

    e2 = jnp.exp(m2 - m1)
    g1 = 1.0 / (1.0 + e2)
    g2 = e2 / (1.0 + e2)
    comb_ref[...] = (jnp.where(col == i1[:, None], g1, 0.0)
                     + jnp.where(col == i2[:, None], g2, 0.0))


def _ffn_body(x_ref, w1_ref, b1_ref, w2_ref, b2_ref, comb_ref, out_ref):
    e = pl.program_id(1)
    x = x_ref[...]
    h = jnp.dot(x, w1_ref[0], preferred_element_type=jnp.float32) + b1_ref[0]
    h = jnp.maximum(h, 0.0)
    y = jnp.dot(h, w2_ref[0], preferred_element_type=jnp.float32) + b2_ref[0]
    comb = comb_ref[...]
    col = jax.lax.broadcasted_iota(jnp.int32, comb.shape, 1)
    c = jnp.sum(jnp.where(col == e, comb, 0.0), axis=1, keepdims=True)
    contrib = y * c

    @pl.when(e == 0)
    def _():
        out_ref[...] = contrib

    @pl.when(e != 0)
    def _():
        out_ref[...] += contrib


def kernel(x, Wg, bg, W1, b1, W2, b2):
    B, D = x.shape
    E = Wg.shape[1]
    H = W1.shape[2]
    BT = min(512, B)
    num_t = B // BT

    probs, idx, comb = pl.pallas_call(
        _gating_body,
        grid=(num_t,),
        in_specs=[
            pl.BlockSpec((BT, D), lambda t: (t, 0)),
            pl.BlockSpec((D, E), lambda t: (0, 0)),
            pl.BlockSpec((1, E), lambda t: (0, 0)),
        ],
        out_specs=[
            pl.BlockSpec((BT, E), lambda t: (t, 0)),
            pl.BlockSpec((BT, 2), lambda t: (t, 0)),
            pl.BlockSpec((BT, E), lambda t: (t, 0)),
        ],
        out_shape=[
            jax.ShapeDtypeStruct((B, E), jnp.float32),
            jax.ShapeDtypeStruct((B, 2), jnp.int32),
            jax.ShapeDtypeStruct((B, E), jnp.float32),
        ],
    )(x, Wg, bg.reshape(1, E))

    out = pl.pallas_call(
        _ffn_body,
        grid=(num_t, E),
        in_specs=[
            pl.BlockSpec((BT, D), lambda t, e: (t, 0)),
            pl.BlockSpec((1, D, H), lambda t, e: (e, 0, 0)),
            pl.BlockSpec((1, 1, H), lambda t, e: (e, 0, 0)),
            pl.BlockSpec((1, H, D), lambda t, e: (e, 0, 0)),
            pl.BlockSpec((1, 1, D), lambda t, e: (e, 0, 0)),
            pl.BlockSpec((BT, E), lambda t, e: (t, 0)),
        ],
        out_specs=pl.BlockSpec((BT, D), lambda t, e: (t, 0)),
        out_shape=jax.ShapeDtypeStruct((B, D), jnp.float32),
    )(x, W1, b1.reshape(E, 1, H), W2, b2.reshape(E, 1, D), comb)

    return out, probs, idx


# speedup vs baseline: 1.7160x; 1.1091x over previous
import functools

import jax
import jax.numpy as jnp
from jax import lax
from jax.experimental import pallas as pl
from jax.experimental.pallas import tpu as pltpu
from jax.experimental.pallas import tpu_sc as plsc


def _gating_body(x_ref, wg_ref, bg_ref,
                 probs_ref, idx_ref, g0t_ref, g1t_ref, lrank_ref, bcount_ref):
    x = x_ref[...]
    logits = jnp.dot(x, wg_ref[...], preferred_element_type=jnp.float32)
    logits = logits + bg_ref[...]
    BT, E = logits.shape
    m1 = jnp.max(logits, axis=1, keepdims=True)
    i1 = jnp.argmax(logits, axis=1)
    col = lax.broadcasted_iota(jnp.int32, (BT, E), 1)
    neg_inf = jnp.float32(-jnp.inf)
    masked = jnp.where(col == i1[:, None], neg_inf, logits)
    m2 = jnp.max(masked, axis=1, keepdims=True)
    i2 = jnp.argmax(masked, axis=1)

    ex = jnp.exp(logits - m1)
    probs_ref[...] = ex / jnp.sum(ex, axis=1, keepdims=True)
    idx_ref[...] = jnp.stack([i1, i2], axis=1)

    e2 = jnp.exp(m2 - m1)
    g1 = 1.0 / (1.0 + e2)
    g2 = e2 / (1.0 + e2)
    g0t_ref[...] = jnp.broadcast_to(g1, (BT, 128))
    g1t_ref[...] = jnp.broadcast_to(g2, (BT, 128))

    oh0 = (col == i1[:, None]).astype(jnp.float32)
    oh1 = (col == i2[:, None]).astype(jnp.float32)
    row = lax.broadcasted_iota(jnp.int32, (BT, BT), 0)
    colt = lax.broadcasted_iota(jnp.int32, (BT, BT), 1)
    tril = (row > colt).astype(jnp.float32)
    s = jax.lax.dot(tril, oh0 + oh1, precision=jax.lax.Precision.HIGHEST,
                    preferred_element_type=jnp.float32)
    r0 = jnp.sum(s * oh0, axis=1, keepdims=True)
    r1 = jnp.sum(s * oh1, axis=1, keepdims=True)
    lrank_ref[...] = jnp.concatenate([r0, r1], axis=1).astype(jnp.int32)
    bcount_ref[...] = jnp.sum(oh0 + oh1, axis=0).astype(jnp.int32).reshape(1, 1, E)


def _make_meta_body(num_tb, bt, blk, nb):
    def _meta_body(bc_ref, idx_ref, lrank_ref, pos_ref, bexp_ref):
        T = num_tb
        bc = bc_ref[...].reshape(T, -1).astype(jnp.float32)
        E = bc.shape[1]
        rt = lax.broadcasted_iota(jnp.int32, (T, T), 0)
        ct = lax.broadcasted_iota(jnp.int32, (T, T), 1)
        trilT = (rt > ct).astype(jnp.float32)
        rank_base = jnp.sum(trilT[:, :, None] * bc[None, :, :], axis=1)
        count = jnp.sum(bc, axis=0, keepdims=True)
        padded = jnp.ceil(count / blk) * blk
        re = lax.broadcasted_iota(jnp.int32, (E, E), 0)
        ce = lax.broadcasted_iota(jnp.int32, (E, E), 1)
        ue = (re <= ce).astype(jnp.float32)
        cum_incl = jnp.sum(padded[0, :, None] * ue, axis=0, keepdims=True)
        seg_start = cum_incl - padded
        base_te = seg_start + rank_base

        idx = idx_ref[...]
        lrank = lrank_ref[...]
        B = idx.shape[0]
        btok = lax.broadcasted_iota(jnp.int32, (B, 1), 0) // bt
        ohtb = (btok == lax.broadcasted_iota(jnp.int32, (B, T), 1)).astype(jnp.float32)
        base_full = jnp.sum(ohtb[:, :, None] * base_te[None, :, :], axis=1)
        colE = lax.broadcasted_iota(jnp.int32, (B, E), 1)
        p0 = jnp.sum(jnp.where(colE == idx[:, 0:1], base_full, 0.0), axis=1,
                     keepdims=True)
        p1 = jnp.sum(jnp.where(colE == idx[:, 1:2], base_full, 0.0), axis=1,
                     keepdims=True)
        pos_ref[...] = jnp.concatenate([p0, p1], axis=1).astype(jnp.int32) + lrank

        rs = (lax.broadcasted_iota(jnp.int32, (nb, 1), 0) * blk).astype(jnp.float32)
        bexp = jnp.sum((cum_incl <= rs).astype(jnp.float32), axis=1, keepdims=True)
        bexp_ref[...] = jnp.minimum(bexp, E - 1).astype(jnp.int32)
    return _meta_body


def _gffn_body(bexp_ref, xs_ref, gs_ref, w1_ref, b1_ref, w2_ref, b2_ref, ys_ref):
    del bexp_ref
    xb = xs_ref[...]
    h = jnp.dot(xb, w1_ref[0], preferred_element_type=jnp.float32) + b1_ref[0]
    h = jnp.maximum(h, 0.0)
    y = jnp.dot(h, w2_ref[0], preferred_element_type=jnp.float32) + b2_ref[0]
    ys_ref[...] = y * gs_ref[:, 0:1]


def kernel(x, Wg, bg, W1, b1, W2, b2):
    B, D = x.shape
    E = Wg.shape[1]
    H = W1.shape[2]
    BT = min(512, B)
    num_tb = B // BT
    BLK = 256
    NB = (2 * B) // BLK + E
    NPAD = NB * BLK

    probs, idx, g0t, g1t, lrank, bcount = pl.pallas_call(
        _gating_body,
        grid=(num_tb,),
        in_specs=[
            pl.BlockSpec((BT, D), lambda t: (t, 0)),
            pl.BlockSpec((D, E), lambda t: (0, 0)),
            pl.BlockSpec((1, E), lambda t: (0, 0)),
        ],
        out_specs=[
            pl.BlockSpec((BT, E), lambda t: (t, 0)),
            pl.BlockSpec((BT, 2), lambda t: (t, 0)),
            pl.BlockSpec((BT, 128), lambda t: (t, 0)),
            pl.BlockSpec((BT, 128), lambda t: (t, 0)),
            pl.BlockSpec((BT, 2), lambda t: (t, 0)),
            pl.BlockSpec((1, 1, E), lambda t: (t, 0, 0)),
        ],
        out_shape=[
            jax.ShapeDtypeStruct((B, E), jnp.float32),
            jax.ShapeDtypeStruct((B, 2), jnp.int32),
            jax.ShapeDtypeStruct((B, 128), jnp.float32),
            jax.ShapeDtypeStruct((B, 128), jnp.float32),
            jax.ShapeDtypeStruct((B, 2), jnp.int32),
            jax.ShapeDtypeStruct((num_tb, 1, E), jnp.int32),
        ],
    )(x, Wg, bg.reshape(1, E))

    pos, bexp = pl.pallas_call(
        _make_meta_body(num_tb, BT, BLK, NB),
        out_shape=[
            jax.ShapeDtypeStruct((B, 2), jnp.int32),
            jax.ShapeDtypeStruct((NB, 1), jnp.int32),
        ],
    )(bcount, idx, lrank)

    p0 = pos[:, 0]
    p1 = pos[:, 1]
    bexp_flat = bexp.reshape(NB)

    info = plsc.get_sparse_core_info()
    NC, NS = info.num_cores, info.num_subcores
    NW = NC * NS
    tok_w = B // NW
    mesh = plsc.VectorSubcoreMesh(core_axis_name="c", subcore_axis_name="s")

    SUB = min(64, tok_w)

    @functools.partial(
        pl.kernel, mesh=mesh,
        out_type=[
            jax.ShapeDtypeStruct((NPAD, D), jnp.float32),
            jax.ShapeDtypeStruct((NPAD, 128), jnp.float32),
        ],
        scratch_types=[
            pltpu.VMEM((SUB, D), jnp.float32),
            pltpu.VMEM((SUB, 128), jnp.float32),
            pltpu.VMEM((SUB, 128), jnp.float32),
            pltpu.VMEM((SUB,), jnp.int32),
            pltpu.VMEM((SUB,), jnp.int32),
            pltpu.SemaphoreType.DMA,
        ],
    )
    def _sc_scatter(x_hbm, p0_hbm, p1_hbm, g0t_hbm, g1t_hbm,
                    xs_hbm, gs_hbm, xbuf, gbuf0, gbuf1, i0, i1, sem):
        wid = lax.axis_index("s") * NC + lax.axis_index("c")
        for sC in range(tok_w // SUB):
            rb = wid * tok_w + sC * SUB
            pltpu.sync_copy(p0_hbm.at[pl.ds(rb, SUB)], i0)
            pltpu.sync_copy(p1_hbm.at[pl.ds(rb, SUB)], i1)
            pltpu.sync_copy(x_hbm.at[pl.ds(rb, SUB)], xbuf)
            pltpu.sync_copy(g0t_hbm.at[pl.ds(rb, SUB)], gbuf0)
            pltpu.sync_copy(g1t_hbm.at[pl.ds(rb, SUB)], gbuf1)
            pltpu.async_copy(xbuf, xs_hbm.at[i0], sem).wait()
            pltpu.async_copy(xbuf, xs_hbm.at[i1], sem).wait()
            pltpu.async_copy(gbuf0, gs_hbm.at[i0], sem).wait()
            pltpu.async_copy(gbuf1, gs_hbm.at[i1], sem).wait()

    xs, gs = _sc_scatter(x, p0, p1, g0t, g1t)

    ys = pl.pallas_call(
        _gffn_body,
        grid_spec=pltpu.PrefetchScalarGridSpec(
            num_scalar_prefetch=1,
            grid=(NB,),
            in_specs=[
                pl.BlockSpec((BLK, D), lambda i, be: (i, 0)),
                pl.BlockSpec((BLK, 128), lambda i, be: (i, 0)),
                pl.BlockSpec((1, D, H), lambda i, be: (be[i], 0, 0)),
                pl.BlockSpec((1, 1, H), lambda i, be: (be[i], 0, 0)),
                pl.BlockSpec((1, H, D), lambda i, be: (be[i], 0, 0)),
                pl.BlockSpec((1, 1, D), lambda i, be: (be[i], 0, 0)),
            ],
            out_specs=pl.BlockSpec((BLK, D), lambda i, be: (i, 0)),
        ),
        out_shape=jax.ShapeDtypeStruct((NPAD, D), jnp.float32),
    )(bexp_flat, xs, gs, W1, b1.reshape(E, 1, H), W2, b2.reshape(E, 1, D))

    CSUB = min(32, tok_w)

    @functools.partial(
        pl.kernel, mesh=mesh,
        out_type=jax.ShapeDtypeStruct((B, D), jnp.float32),
        scratch_types=[
            pltpu.VMEM((CSUB, D), jnp.float32),
            pltpu.VMEM((CSUB, D), jnp.float32),
            pltpu.VMEM((CSUB,), jnp.int32),
            pltpu.VMEM((CSUB,), jnp.int32),
            pltpu.SemaphoreType.DMA,
        ],
    )
    def _sc_combine(ys_hbm, p0_hbm, p1_hbm, out_hbm, y0, y1, i0, i1, sem):
        wid = lax.axis_index("s") * NC + lax.axis_index("c")
        for sC in range(tok_w // CSUB):
            rb = wid * tok_w + sC * CSUB
            pltpu.sync_copy(p0_hbm.at[pl.ds(rb, CSUB)], i0)
            pltpu.sync_copy(p1_hbm.at[pl.ds(rb, CSUB)], i1)
            cp0 = pltpu.async_copy(ys_hbm.at[i0], y0, sem)
            cp1 = pltpu.async_copy(ys_hbm.at[i1], y1, sem)
            cp0.wait()
            cp1.wait()

            def body(t, carry):
                for v in range(D // 16):
                    sl = pl.ds(v * 16, 16)
                    y0[t, sl] = y0[t, sl] + y1[t, sl]
                return carry

            lax.fori_loop(0, CSUB, body, 0)
            pltpu.sync_copy(y0, out_hbm.at[pl.ds(rb, CSUB)])

    out = _sc_combine(ys, p0, p1)
    return out, probs, idx


# speedup vs baseline: 1.8372x; 1.0706x over previous
import functools

import jax
import jax.numpy as jnp
from jax import lax
from jax.experimental import pallas as pl
from jax.experimental.pallas import tpu as pltpu
from jax.experimental.pallas import tpu_sc as plsc


def _gating_body(x_ref, wg_ref, bg_ref,
                 probs_ref, idx_ref, g0t_ref, g1t_ref, lrank_ref, bcount_ref,
                 xpk_ref):
    x = x_ref[...]
    D2 = x.shape[1] // 2
    l16 = lax.bitcast_convert_type(x[:, :D2].astype(jnp.bfloat16), jnp.uint16)
    h16 = lax.bitcast_convert_type(x[:, D2:].astype(jnp.bfloat16), jnp.uint16)
    w = l16.astype(jnp.uint32) | (h16.astype(jnp.uint32) << 16)
    xpk_ref[...] = lax.bitcast_convert_type(w, jnp.float32)
    logits = jnp.dot(x, wg_ref[...], preferred_element_type=jnp.float32)
    logits = logits + bg_ref[...]
    BT, E = logits.shape
    m1 = jnp.max(logits, axis=1, keepdims=True)
    i1 = jnp.argmax(logits, axis=1)
    col = lax.broadcasted_iota(jnp.int32, (BT, E), 1)
    neg_inf = jnp.float32(-jnp.inf)
    masked = jnp.where(col == i1[:, None], neg_inf, logits)
    m2 = jnp.max(masked, axis=1, keepdims=True)
    i2 = jnp.argmax(masked, axis=1)

    ex = jnp.exp(logits - m1)
    probs_ref[...] = ex / jnp.sum(ex, axis=1, keepdims=True)
    idx_ref[...] = jnp.stack([i1, i2], axis=1)

    e2 = jnp.exp(m2 - m1)
    g1 = 1.0 / (1.0 + e2)
    g2 = e2 / (1.0 + e2)
    g0t_ref[...] = jnp.broadcast_to(g1, (BT, 128))
    g1t_ref[...] = jnp.broadcast_to(g2, (BT, 128))

    oh0 = (col == i1[:, None]).astype(jnp.float32)
    oh1 = (col == i2[:, None]).astype(jnp.float32)
    row = lax.broadcasted_iota(jnp.int32, (BT, BT), 0)
    colt = lax.broadcasted_iota(jnp.int32, (BT, BT), 1)
    tril = (row > colt).astype(jnp.float32)
    s = jax.lax.dot(tril, oh0 + oh1, precision=jax.lax.Precision.HIGHEST,
                    preferred_element_type=jnp.float32)
    r0 = jnp.sum(s * oh0, axis=1, keepdims=True)
    r1 = jnp.sum(s * oh1, axis=1, keepdims=True)
    lrank_ref[...] = jnp.concatenate([r0, r1], axis=1).astype(jnp.int32)
    bcount_ref[...] = jnp.sum(oh0 + oh1, axis=0).astype(jnp.int32).reshape(1, 1, E)


def _make_meta_body(num_tb, bt, blk, nb):
    def _meta_body(bc_ref, idx_ref, lrank_ref, pos_ref, bexp_ref):
        T = num_tb
        bc = bc_ref[...].reshape(T, -1).astype(jnp.float32)
        E = bc.shape[1]
        rt = lax.broadcasted_iota(jnp.int32, (T, T), 0)
        ct = lax.broadcasted_iota(jnp.int32, (T, T), 1)
        trilT = (rt > ct).astype(jnp.float32)
        rank_base = jnp.sum(trilT[:, :, None] * bc[None, :, :], axis=1)
        count = jnp.sum(bc, axis=0, keepdims=True)
        padded = jnp.ceil(count / blk) * blk
        re = lax.broadcasted_iota(jnp.int32, (E, E), 0)
        ce = lax.broadcasted_iota(jnp.int32, (E, E), 1)
        ue = (re <= ce).astype(jnp.float32)
        cum_incl = jnp.sum(padded[0, :, None] * ue, axis=0, keepdims=True)
        seg_start = cum_incl - padded
        base_te = seg_start + rank_base

        idx = idx_ref[...]
        lrank = lrank_ref[...]
        B = idx.shape[0]
        btok = lax.broadcasted_iota(jnp.int32, (B, 1), 0) // bt
        ohtb = (btok == lax.broadcasted_iota(jnp.int32, (B, T), 1)).astype(jnp.float32)
        base_full = jnp.sum(ohtb[:, :, None] * base_te[None, :, :], axis=1)
        colE = lax.broadcasted_iota(jnp.int32, (B, E), 1)
        p0 = jnp.sum(jnp.where(colE == idx[:, 0:1], base_full, 0.0), axis=1,
                     keepdims=True)
        p1 = jnp.sum(jnp.where(colE == idx[:, 1:2], base_full, 0.0), axis=1,
                     keepdims=True)
        pos_ref[...] = jnp.concatenate([p0, p1], axis=1).astype(jnp.int32) + lrank

        rs = (lax.broadcasted_iota(jnp.int32, (nb, 1), 0) * blk).astype(jnp.float32)
        bexp = jnp.sum((cum_incl <= rs).astype(jnp.float32), axis=1, keepdims=True)
        bexp_ref[...] = jnp.minimum(bexp, E - 1).astype(jnp.int32)
    return _meta_body


def _gffn_body(bexp_ref, xs_ref, gs_ref, w1_ref, b1_ref, w2_ref, b2_ref, ys_ref):
    del bexp_ref
    w = lax.bitcast_convert_type(xs_ref[...], jnp.uint32)
    D2 = w.shape[1]
    lo = lax.bitcast_convert_type(w << 16, jnp.float32)
    hi = lax.bitcast_convert_type(w & jnp.uint32(0xFFFF0000), jnp.float32)
    h = (jnp.dot(lo, w1_ref[0, :D2, :], preferred_element_type=jnp.float32)
         + jnp.dot(hi, w1_ref[0, D2:, :], preferred_element_type=jnp.float32)
         + b1_ref[0])
    h = jnp.maximum(h, 0.0)
    y = jnp.dot(h, w2_ref[0], preferred_element_type=jnp.float32) + b2_ref[0]
    ys_ref[...] = y * gs_ref[:, 0:1]


def kernel(x, Wg, bg, W1, b1, W2, b2):
    B, D = x.shape
    E = Wg.shape[1]
    H = W1.shape[2]
    BT = min(512, B)
    num_tb = B // BT
    BLK = 256
    NB = (2 * B) // BLK + E
    NPAD = NB * BLK

    probs, idx, g0t, g1t, lrank, bcount, xpk = pl.pallas_call(
        _gating_body,
        grid=(num_tb,),
        in_specs=[
            pl.BlockSpec((BT, D), lambda t: (t, 0)),
            pl.BlockSpec((D, E), lambda t: (0, 0)),
            pl.BlockSpec((1, E), lambda t: (0, 0)),
        ],
        out_specs=[
            pl.BlockSpec((BT, E), lambda t: (t, 0)),
            pl.BlockSpec((BT, 2), lambda t: (t, 0)),
            pl.BlockSpec((BT, 128), lambda t: (t, 0)),
            pl.BlockSpec((BT, 128), lambda t: (t, 0)),
            pl.BlockSpec((BT, 2), lambda t: (t, 0)),
            pl.BlockSpec((1, 1, E), lambda t: (t, 0, 0)),
            pl.BlockSpec((BT, D // 2), lambda t: (t, 0)),
        ],
        out_shape=[
            jax.ShapeDtypeStruct((B, E), jnp.float32),
            jax.ShapeDtypeStruct((B, 2), jnp.int32),
            jax.ShapeDtypeStruct((B, 128), jnp.float32),
            jax.ShapeDtypeStruct((B, 128), jnp.float32),
            jax.ShapeDtypeStruct((B, 2), jnp.int32),
            jax.ShapeDtypeStruct((num_tb, 1, E), jnp.int32),
            jax.ShapeDtypeStruct((B, D // 2), jnp.float32),
        ],
    )(x, Wg, bg.reshape(1, E))

    pos, bexp = pl.pallas_call(
        _make_meta_body(num_tb, BT, BLK, NB),
        out_shape=[
            jax.ShapeDtypeStruct((B, 2), jnp.int32),
            jax.ShapeDtypeStruct((NB, 1), jnp.int32),
        ],
    )(bcount, idx, lrank)

    p0 = pos[:, 0]
    p1 = pos[:, 1]
    bexp_flat = bexp.reshape(NB)

    info = plsc.get_sparse_core_info()
    NC, NS = info.num_cores, info.num_subcores
    NW = NC * NS
    tok_w = B // NW
    mesh = plsc.VectorSubcoreMesh(core_axis_name="c", subcore_axis_name="s")

    SUB = min(64, tok_w)

    @functools.partial(
        pl.kernel, mesh=mesh,
        out_type=[
            jax.ShapeDtypeStruct((NPAD, D // 2), jnp.float32),
            jax.ShapeDtypeStruct((NPAD, 128), jnp.float32),
        ],
        scratch_types=[
            pltpu.VMEM((SUB, D // 2), jnp.float32),
            pltpu.VMEM((SUB, 128), jnp.float32),
            pltpu.VMEM((SUB, 128), jnp.float32),
            pltpu.VMEM((SUB,), jnp.int32),
            pltpu.VMEM((SUB,), jnp.int32),
            pltpu.SemaphoreType.DMA,
        ],
    )
    def _sc_scatter(x_hbm, p0_hbm, p1_hbm, g0t_hbm, g1t_hbm,
                    xs_hbm, gs_hbm, xbuf, gbuf0, gbuf1, i0, i1, sem):
        wid = lax.axis_index("s") * NC + lax.axis_index("c")
        for sC in range(tok_w // SUB):
            rb = wid * tok_w + sC * SUB
            lds = [
                pltpu.async_copy(p0_hbm.at[pl.ds(rb, SUB)], i0, sem),
                pltpu.async_copy(p1_hbm.at[pl.ds(rb, SUB)], i1, sem),
                pltpu.async_copy(x_hbm.at[pl.ds(rb, SUB)], xbuf, sem),
                pltpu.async_copy(g0t_hbm.at[pl.ds(rb, SUB)], gbuf0, sem),
                pltpu.async_copy(g1t_hbm.at[pl.ds(rb, SUB)], gbuf1, sem),
            ]
            for c in lds:
                c.wait()
            sts = [
                pltpu.async_copy(xbuf, xs_hbm.at[i0], sem),
                pltpu.async_copy(xbuf, xs_hbm.at[i1], sem),
                pltpu.async_copy(gbuf0, gs_hbm.at[i0], sem),
                pltpu.async_copy(gbuf1, gs_hbm.at[i1], sem),
            ]
            for c in sts:
                c.wait()

    xs, gs = _sc_scatter(xpk, p0, p1, g0t, g1t)

    ys = pl.pallas_call(
        _gffn_body,
        grid_spec=pltpu.PrefetchScalarGridSpec(
            num_scalar_prefetch=1,
            grid=(NB,),
            in_specs=[
                pl.BlockSpec((BLK, D // 2), lambda i, be: (i, 0)),
                pl.BlockSpec((BLK, 128), lambda i, be: (i, 0)),
                pl.BlockSpec((1, D, H), lambda i, be: (be[i], 0, 0)),
                pl.BlockSpec((1, 1, H), lambda i, be: (be[i], 0, 0)),
                pl.BlockSpec((1, H, D), lambda i, be: (be[i], 0, 0)),
                pl.BlockSpec((1, 1, D), lambda i, be: (be[i], 0, 0)),
            ],
            out_specs=pl.BlockSpec((BLK, D), lambda i, be: (i, 0)),
        ),
        out_shape=jax.ShapeDtypeStruct((NPAD, D), jnp.float32),
    )(bexp_flat, xs, gs, W1, b1.reshape(E, 1, H), W2, b2.reshape(E, 1, D))

    CSUB = min(32, tok_w)

    @functools.partial(
        pl.kernel, mesh=mesh,
        out_type=jax.ShapeDtypeStruct((B, D), jnp.float32),
        scratch_types=[
            pltpu.VMEM((CSUB, D), jnp.float32),
            pltpu.VMEM((CSUB, D), jnp.float32),
            pltpu.VMEM((CSUB,), jnp.int32),
            pltpu.VMEM((CSUB,), jnp.int32),
            pltpu.SemaphoreType.DMA,
        ],
    )
    def _sc_combine(ys_hbm, p0_hbm, p1_hbm, out_hbm, y0, y1, i0, i1, sem):
        wid = lax.axis_index("s") * NC + lax.axis_index("c")
        for sC in range(tok_w // CSUB):
            rb = wid * tok_w + sC * CSUB
            pltpu.sync_copy(p0_hbm.at[pl.ds(rb, CSUB)], i0)
            pltpu.sync_copy(p1_hbm.at[pl.ds(rb, CSUB)], i1)
            cp0 = pltpu.async_copy(ys_hbm.at[i0], y0, sem)
            cp1 = pltpu.async_copy(ys_hbm.at[i1], y1, sem)
            cp0.wait()
            cp1.wait()

            def body(t, carry):
                for v in range(D // 16):
                    sl = pl.ds(v * 16, 16)
                    y0[t, sl] = y0[t, sl] + y1[t, sl]
                return carry

            lax.fori_loop(0, CSUB, body, 0)
            pltpu.sync_copy(y0, out_hbm.at[pl.ds(rb, CSUB)])

    out = _sc_combine(ys, p0, p1)
    return out, probs, idx


# speedup vs baseline: 1.9663x; 1.0703x over previous
import functools

import jax
import jax.numpy as jnp
from jax import lax
from jax.experimental import pallas as pl
from jax.experimental.pallas import tpu as pltpu
from jax.experimental.pallas import tpu_sc as plsc


def _gating_body(x_ref, wg_ref, bg_ref,
                 probs_ref, idx_ref, g0t_ref, g1t_ref, lrank_ref, bcount_ref,
                 xpk_ref):
    x = x_ref[...]
    D2 = x.shape[1] // 2
    l16 = lax.bitcast_convert_type(x[:, :D2].astype(jnp.bfloat16), jnp.uint16)
    h16 = lax.bitcast_convert_type(x[:, D2:].astype(jnp.bfloat16), jnp.uint16)
    w = l16.astype(jnp.uint32) | (h16.astype(jnp.uint32) << 16)
    xpk_ref[...] = lax.bitcast_convert_type(w, jnp.float32)
    logits = jnp.dot(x, wg_ref[...], preferred_element_type=jnp.float32)
    logits = logits + bg_ref[...]
    BT, E = logits.shape
    m1 = jnp.max(logits, axis=1, keepdims=True)
    i1 = jnp.argmax(logits, axis=1)
    col = lax.broadcasted_iota(jnp.int32, (BT, E), 1)
    neg_inf = jnp.float32(-jnp.inf)
    masked = jnp.where(col == i1[:, None], neg_inf, logits)
    m2 = jnp.max(masked, axis=1, keepdims=True)
    i2 = jnp.argmax(masked, axis=1)

    ex = jnp.exp(logits - m1)
    probs_ref[...] = ex / jnp.sum(ex, axis=1, keepdims=True)
    idx_ref[...] = jnp.stack([i1, i2], axis=1)

    e2 = jnp.exp(m2 - m1)
    g1 = 1.0 / (1.0 + e2)
    g2 = e2 / (1.0 + e2)
    g0t_ref[...] = jnp.broadcast_to(g1, (BT, 128))
    g1t_ref[...] = jnp.broadcast_to(g2, (BT, 128))

    oh0 = (col == i1[:, None]).astype(jnp.float32)
    oh1 = (col == i2[:, None]).astype(jnp.float32)
    row = lax.broadcasted_iota(jnp.int32, (BT, BT), 0)
    colt = lax.broadcasted_iota(jnp.int32, (BT, BT), 1)
    tril = (row > colt).astype(jnp.float32)
    s = jax.lax.dot(tril, oh0 + oh1, precision=jax.lax.Precision.HIGHEST,
                    preferred_element_type=jnp.float32)
    r0 = jnp.sum(s * oh0, axis=1, keepdims=True)
    r1 = jnp.sum(s * oh1, axis=1, keepdims=True)
    lrank_ref[...] = jnp.concatenate([r0, r1], axis=1).astype(jnp.int32)
    bcount_ref[...] = jnp.sum(oh0 + oh1, axis=0).astype(jnp.int32).reshape(1, 1, E)


def _make_meta_body(num_tb, bt, blk, nb):
    def _meta_body(bc_ref, idx_ref, lrank_ref, pos_ref, bexp_ref):
        T = num_tb
        bc = bc_ref[...].reshape(T, -1).astype(jnp.float32)
        E = bc.shape[1]
        rt = lax.broadcasted_iota(jnp.int32, (T, T), 0)
        ct = lax.broadcasted_iota(jnp.int32, (T, T), 1)
        trilT = (rt > ct).astype(jnp.float32)
        rank_base = jnp.sum(trilT[:, :, None] * bc[None, :, :], axis=1)
        count = jnp.sum(bc, axis=0, keepdims=True)
        padded = jnp.ceil(count / blk) * blk
        re = lax.broadcasted_iota(jnp.int32, (E, E), 0)
        ce = lax.broadcasted_iota(jnp.int32, (E, E), 1)
        ue = (re <= ce).astype(jnp.float32)
        cum_incl = jnp.sum(padded[0, :, None] * ue, axis=0, keepdims=True)
        seg_start = cum_incl - padded
        base_te = seg_start + rank_base

        idx = idx_ref[...]
        lrank = lrank_ref[...]
        B = idx.shape[0]
        btok = lax.broadcasted_iota(jnp.int32, (B, 1), 0) // bt
        ohtb = (btok == lax.broadcasted_iota(jnp.int32, (B, T), 1)).astype(jnp.float32)
        base_full = jnp.sum(ohtb[:, :, None] * base_te[None, :, :], axis=1)
        colE = lax.broadcasted_iota(jnp.int32, (B, E), 1)
        p0 = jnp.sum(jnp.where(colE == idx[:, 0:1], base_full, 0.0), axis=1,
                     keepdims=True)
        p1 = jnp.sum(jnp.where(colE == idx[:, 1:2], base_full, 0.0), axis=1,
                     keepdims=True)
        pos_ref[...] = jnp.concatenate([p0, p1], axis=1).astype(jnp.int32) + lrank

        rs = (lax.broadcasted_iota(jnp.int32, (nb, 1), 0) * blk).astype(jnp.float32)
        bexp = jnp.sum((cum_incl <= rs).astype(jnp.float32), axis=1, keepdims=True)
        bexp_ref[...] = jnp.minimum(bexp, E - 1).astype(jnp.int32)
    return _meta_body


def _gffn_body(bexp_ref, xs_ref, gs_ref, w1_ref, b1_ref, w2_ref, b2_ref, ys_ref):
    del bexp_ref
    w = lax.bitcast_convert_type(xs_ref[...], jnp.uint32)
    D2 = w.shape[1]
    lo = lax.bitcast_convert_type(w << 16, jnp.float32)
    hi = lax.bitcast_convert_type(w & jnp.uint32(0xFFFF0000), jnp.float32)
    h = (jnp.dot(lo, w1_ref[0, :D2, :], preferred_element_type=jnp.float32)
         + jnp.dot(hi, w1_ref[0, D2:, :], preferred_element_type=jnp.float32)
         + b1_ref[0])
    h = jnp.maximum(h, 0.0)
    y = jnp.dot(h, w2_ref[0], preferred_element_type=jnp.float32) + b2_ref[0]
    y = y * gs_ref[:, 0:1]
    l16 = lax.bitcast_convert_type(y[:, :D2].astype(jnp.bfloat16), jnp.uint16)
    h16 = lax.bitcast_convert_type(y[:, D2:].astype(jnp.bfloat16), jnp.uint16)
    wo = l16.astype(jnp.uint32) | (h16.astype(jnp.uint32) << 16)
    ys_ref[...] = lax.bitcast_convert_type(wo, jnp.float32)


def _unpack_add_body(y0_ref, y1_ref, out_ref):
    w0 = lax.bitcast_convert_type(y0_ref[...], jnp.uint32)
    w1 = lax.bitcast_convert_type(y1_ref[...], jnp.uint32)
    msk = jnp.uint32(0xFFFF0000)
    lo = (lax.bitcast_convert_type(w0 << 16, jnp.float32)
          + lax.bitcast_convert_type(w1 << 16, jnp.float32))
    hi = (lax.bitcast_convert_type(w0 & msk, jnp.float32)
          + lax.bitcast_convert_type(w1 & msk, jnp.float32))
    out_ref[...] = jnp.concatenate([lo, hi], axis=1)


def kernel(x, Wg, bg, W1, b1, W2, b2):
    B, D = x.shape
    E = Wg.shape[1]
    H = W1.shape[2]
    BT = min(512, B)
    num_tb = B // BT
    BLK = 256
    NB = (2 * B) // BLK + E
    NPAD = NB * BLK

    probs, idx, g0t, g1t, lrank, bcount, xpk = pl.pallas_call(
        _gating_body,
        grid=(num_tb,),
        in_specs=[
            pl.BlockSpec((BT, D), lambda t: (t, 0)),
            pl.BlockSpec((D, E), lambda t: (0, 0)),
            pl.BlockSpec((1, E), lambda t: (0, 0)),
        ],
        out_specs=[
            pl.BlockSpec((BT, E), lambda t: (t, 0)),
            pl.BlockSpec((BT, 2), lambda t: (t, 0)),
            pl.BlockSpec((BT, 128), lambda t: (t, 0)),
            pl.BlockSpec((BT, 128), lambda t: (t, 0)),
            pl.BlockSpec((BT, 2), lambda t: (t, 0)),
            pl.BlockSpec((1, 1, E), lambda t: (t, 0, 0)),
            pl.BlockSpec((BT, D // 2), lambda t: (t, 0)),
        ],
        out_shape=[
            jax.ShapeDtypeStruct((B, E), jnp.float32),
            jax.ShapeDtypeStruct((B, 2), jnp.int32),
            jax.ShapeDtypeStruct((B, 128), jnp.float32),
            jax.ShapeDtypeStruct((B, 128), jnp.float32),
            jax.ShapeDtypeStruct((B, 2), jnp.int32),
            jax.ShapeDtypeStruct((num_tb, 1, E), jnp.int32),
            jax.ShapeDtypeStruct((B, D // 2), jnp.float32),
        ],
    )(x, Wg, bg.reshape(1, E))

    pos, bexp = pl.pallas_call(
        _make_meta_body(num_tb, BT, BLK, NB),
        out_shape=[
            jax.ShapeDtypeStruct((B, 2), jnp.int32),
            jax.ShapeDtypeStruct((NB, 1), jnp.int32),
        ],
    )(bcount, idx, lrank)

    p0 = pos[:, 0]
    p1 = pos[:, 1]
    bexp_flat = bexp.reshape(NB)

    info = plsc.get_sparse_core_info()
    NC, NS = info.num_cores, info.num_subcores
    NW = NC * NS
    tok_w = B // NW
    mesh = plsc.VectorSubcoreMesh(core_axis_name="c", subcore_axis_name="s")

    SUB = min(64, tok_w)

    @functools.partial(
        pl.kernel, mesh=mesh,
        out_type=[
            jax.ShapeDtypeStruct((NPAD, D // 2), jnp.float32),
            jax.ShapeDtypeStruct((NPAD, 128), jnp.float32),
        ],
        scratch_types=[
            pltpu.VMEM((SUB, D // 2), jnp.float32),
            pltpu.VMEM((SUB, 128), jnp.float32),
            pltpu.VMEM((SUB, 128), jnp.float32),
            pltpu.VMEM((SUB,), jnp.int32),
            pltpu.VMEM((SUB,), jnp.int32),
            pltpu.SemaphoreType.DMA,
        ],
    )
    def _sc_scatter(x_hbm, p0_hbm, p1_hbm, g0t_hbm, g1t_hbm,
                    xs_hbm, gs_hbm, xbuf, gbuf0, gbuf1, i0, i1, sem):
        wid = lax.axis_index("s") * NC + lax.axis_index("c")
        for sC in range(tok_w // SUB):
            rb = wid * tok_w + sC * SUB
            lds = [
                pltpu.async_copy(p0_hbm.at[pl.ds(rb, SUB)], i0, sem),
                pltpu.async_copy(p1_hbm.at[pl.ds(rb, SUB)], i1, sem),
                pltpu.async_copy(x_hbm.at[pl.ds(rb, SUB)], xbuf, sem),
                pltpu.async_copy(g0t_hbm.at[pl.ds(rb, SUB)], gbuf0, sem),
                pltpu.async_copy(g1t_hbm.at[pl.ds(rb, SUB)], gbuf1, sem),
            ]
            for c in lds:
                c.wait()
            sts = [
                pltpu.async_copy(xbuf, xs_hbm.at[i0], sem),
                pltpu.async_copy(xbuf, xs_hbm.at[i1], sem),
                pltpu.async_copy(gbuf0, gs_hbm.at[i0], sem),
                pltpu.async_copy(gbuf1, gs_hbm.at[i1], sem),
            ]
            for c in sts:
                c.wait()

    xs, gs = _sc_scatter(xpk, p0, p1, g0t, g1t)

    ys = pl.pallas_call(
        _gffn_body,
        grid_spec=pltpu.PrefetchScalarGridSpec(
            num_scalar_prefetch=1,
            grid=(NB,),
            in_specs=[
                pl.BlockSpec((BLK, D // 2), lambda i, be: (i, 0)),
                pl.BlockSpec((BLK, 128), lambda i, be: (i, 0)),
                pl.BlockSpec((1, D, H), lambda i, be: (be[i], 0, 0)),
                pl.BlockSpec((1, 1, H), lambda i, be: (be[i], 0, 0)),
                pl.BlockSpec((1, H, D), lambda i, be: (be[i], 0, 0)),
                pl.BlockSpec((1, 1, D), lambda i, be: (be[i], 0, 0)),
            ],
            out_specs=pl.BlockSpec((BLK, D // 2), lambda i, be: (i, 0)),
        ),
        out_shape=jax.ShapeDtypeStruct((NPAD, D // 2), jnp.float32),
    )(bexp_flat, xs, gs, W1, b1.reshape(E, 1, H), W2, b2.reshape(E, 1, D))

    CSUB = min(64, tok_w)

    @functools.partial(
        pl.kernel, mesh=mesh,
        out_type=[
            jax.ShapeDtypeStruct((B, D // 2), jnp.float32),
            jax.ShapeDtypeStruct((B, D // 2), jnp.float32),
        ],
        scratch_types=[
            pltpu.VMEM((CSUB, D // 2), jnp.float32),
            pltpu.VMEM((CSUB, D // 2), jnp.float32),
            pltpu.VMEM((CSUB,), jnp.int32),
            pltpu.VMEM((CSUB,), jnp.int32),
            pltpu.SemaphoreType.DMA,
        ],
    )
    def _sc_combine(ys_hbm, p0_hbm, p1_hbm, yg0_hbm, yg1_hbm,
                    y0, y1, i0, i1, sem):
        wid = lax.axis_index("s") * NC + lax.axis_index("c")
        for sC in range(tok_w // CSUB):
            rb = wid * tok_w + sC * CSUB
            ca = pltpu.async_copy(p0_hbm.at[pl.ds(rb, CSUB)], i0, sem)
            cb = pltpu.async_copy(p1_hbm.at[pl.ds(rb, CSUB)], i1, sem)
            ca.wait()
            cb.wait()
            cp0 = pltpu.async_copy(ys_hbm.at[i0], y0, sem)
            cp1 = pltpu.async_copy(ys_hbm.at[i1], y1, sem)
            cp0.wait()
            cp1.wait()
            co0 = pltpu.async_copy(y0, yg0_hbm.at[pl.ds(rb, CSUB)], sem)
            co1 = pltpu.async_copy(y1, yg1_hbm.at[pl.ds(rb, CSUB)], sem)
            co0.wait()
            co1.wait()

    yg0, yg1 = _sc_combine(ys, p0, p1)

    BT2 = min(512, B)
    out = pl.pallas_call(
        _unpack_add_body,
        grid=(B // BT2,),
        in_specs=[
            pl.BlockSpec((BT2, D // 2), lambda t: (t, 0)),
            pl.BlockSpec((BT2, D // 2), lambda t: (t, 0)),
        ],
        out_specs=pl.BlockSpec((BT2, D), lambda t: (t, 0)),
        out_shape=jax.ShapeDtypeStruct((B, D), jnp.float32),
    )(yg0, yg1)
    return out, probs, idx
